# Initial kernel scaffold; baseline (speedup 1.0000x reference)
import jax
import jax.numpy as jnp
from jax.experimental import pallas as pl


def kernel(inputs, idxs):
    raise NotImplementedError("write your pallas kernel here")


# speedup vs baseline: 1.3732x; 1.3732x over previous
import functools

import jax
import jax.numpy as jnp
from jax import lax
from jax.experimental import pallas as pl
from jax.experimental.pallas import tpu as pltpu
from jax.experimental.pallas import tpu_sc as plsc

_NC = 2
_NS = 16
_NW = _NC * _NS

_IB = 128


def _sc_gather(x, idxs2, B, S2, C):
    BS2 = B * S2
    RW = BS2 // _NW
    NIR = RW // _IB
    WPB = _NW // B
    CH = 512
    NCH = RW // CH
    JPC = CH // _IB

    mesh = plsc.VectorSubcoreMesh(core_axis_name="c", subcore_axis_name="s")

    @functools.partial(
        pl.kernel,
        mesh=mesh,
        compiler_params=pltpu.CompilerParams(use_tc_tiling_on_sc=False),
        out_type=jax.ShapeDtypeStruct((BS2, C), jnp.float32),
        scratch_types=[
            pltpu.VMEM((NIR, _IB), jnp.int32),
            pltpu.VMEM((CH, C), jnp.float32),
            pltpu.SemaphoreType.DMA,
        ],
    )
    def k(x_hbm, idxs_hbm, out_hbm, idx_v, rows_v, sem):
        wid = lax.axis_index("s") * _NC + lax.axis_index("c")
        base = wid * RW
        b = wid // WPB
        boff = b * S2
        irow0 = (wid % WPB) * NIR

        pltpu.sync_copy(idxs_hbm.at[pl.ds(irow0, NIR)], idx_v)

        def add_body(r, carry):
            for i in range(_IB // 16):
                sl = pl.ds(i * 16, 16)
                idx_v[r, sl] = idx_v[r, sl] + boff
            return carry

        lax.fori_loop(0, NIR, add_body, 0)

        def chunk_body(g, carry):
            r0 = g * JPC
            cps = [
                pltpu.async_copy(
                    x_hbm.at[idx_v.at[r0 + j]],
                    rows_v.at[pl.ds(j * _IB, _IB)],
                    sem,
                )
                for j in range(JPC)
            ]
            for cp in cps:
                cp.wait()
            pltpu.sync_copy(rows_v, out_hbm.at[pl.ds(base + g * CH, CH)])
            return carry

        lax.fori_loop(0, NCH, chunk_body, 0)

    return k(x, idxs2)


def kernel(inputs, idxs):
    B, S, _, C = inputs.shape
    S2 = S * S
    x = inputs.reshape(B * S2, C)
    idxs2 = idxs.reshape(S2 // _IB, _IB)
    out = _sc_gather(x, idxs2, B, S2, C)
    return out.reshape(B, S2, C)


# speedup vs baseline: 1.3842x; 1.0080x over previous
import functools

import jax
import jax.numpy as jnp
from jax import lax
from jax.experimental import pallas as pl
from jax.experimental.pallas import tpu as pltpu
from jax.experimental.pallas import tpu_sc as plsc

_NC = 2
_NS = 16
_NW = _NC * _NS

_IB = 128


def _sc_gather(x, idxs2, B, S2, C):
    BS2 = B * S2
    RW = BS2 // _NW
    NIR = RW // _IB
    WPB = _NW // B
    CH = 512
    NCH = RW // CH
    JPC = CH // _IB
    NBUF = 2
    WBYTES = CH * C * 4
    GBYTES = _IB * C * 4

    mesh = plsc.VectorSubcoreMesh(core_axis_name="c", subcore_axis_name="s")

    @functools.partial(
        pl.kernel,
        mesh=mesh,
        compiler_params=pltpu.CompilerParams(use_tc_tiling_on_sc=False),
        out_type=jax.ShapeDtypeStruct((BS2, C), jnp.float32),
        scratch_types=[
            pltpu.VMEM((NIR, _IB), jnp.int32),
            pltpu.VMEM((NBUF, CH, C), jnp.float32),
            [pltpu.SemaphoreType.DMA] * NBUF,
            [pltpu.SemaphoreType.DMA] * NBUF,
        ],
    )
    def k(x_hbm, idxs_hbm, out_hbm, idx_v, rows_v, gsems, wsems):
        wid = lax.axis_index("s") * _NC + lax.axis_index("c")
        base = wid * RW
        b = wid // WPB
        boff = b * S2
        irow0 = (wid % WPB) * NIR

        pltpu.sync_copy(idxs_hbm.at[pl.ds(irow0, NIR)], idx_v)

        def add_body(r, carry):
            for i in range(_IB // 16):
                sl = pl.ds(i * 16, 16)
                idx_v[r, sl] = idx_v[r, sl] + boff
            return carry

        lax.fori_loop(0, NIR, add_body, 0)

        def fire_gather(g, buf):
            r0 = g * JPC
            for j in range(JPC):
                pltpu.async_copy(
                    x_hbm.at[idx_v.at[r0 + j]],
                    rows_v.at[buf, pl.ds(j * _IB, _IB)],
                    gsems[buf],
                )

        def wait_gather(buf):
            for j in range(JPC):
                pltpu.make_async_copy(
                    x_hbm.at[pl.ds(0, _IB)],
                    rows_v.at[buf, pl.ds(j * _IB, _IB)],
                    gsems[buf],
                ).wait()

        def fire_write(g, buf):
            pltpu.async_copy(
                rows_v.at[buf],
                out_hbm.at[pl.ds(base + g * CH, CH)],
                wsems[buf],
            )

        def wait_write(buf):
            pltpu.make_async_copy(
                rows_v.at[buf], out_hbm.at[pl.ds(0, CH)], wsems[buf]
            ).wait()

        NITER = NCH // NBUF
        for buf in range(NBUF):
            fire_gather(buf, buf)

        def chunk_body(i, carry):
            for buf in range(NBUF):
                wait_gather(buf)
                fire_write(i * NBUF + buf, buf)
            for buf in range(NBUF):
                wait_write(buf)
                fire_gather((i + 1) * NBUF + buf, buf)
            return carry

        lax.fori_loop(0, NITER - 1, chunk_body, 0)

        for buf in range(NBUF):
            wait_gather(buf)
            fire_write((NITER - 1) * NBUF + buf, buf)
        for buf in range(NBUF):
            wait_write(buf)

    return k(x, idxs2)


def kernel(inputs, idxs):
    B, S, _, C = inputs.shape
    S2 = S * S
    x = inputs.reshape(B * S2, C)
    idxs2 = idxs.reshape(S2 // _IB, _IB)
    out = _sc_gather(x, idxs2, B, S2, C)
    return out.reshape(B, S2, C)
